# baseline (speedup 1.0000x reference)
import functools

import jax
import jax.numpy as jnp
from jax import lax
from jax.experimental import pallas as pl
from jax.experimental.pallas import tpu as pltpu
from jax.experimental.pallas import tpu_sc as plsc

NC = 2
NS = 16
L = 16

CHUNK = 80
WH = 64


def _sc_edge_pass(table2, al, ar, src4, dst4):
    n = table2.shape[0] // 2
    nch = src4.shape[2]
    rows_per_tile = n // NS
    nz = rows_per_tile // CHUNK
    zrem = rows_per_tile % CHUNK
    nj = WH // L
    ng = CHUNK // L

    mesh = plsc.VectorSubcoreMesh(
        core_axis_name="c", subcore_axis_name="s",
        num_cores=NC, num_subcores=NS)

    @functools.partial(
        pl.kernel,
        out_type=(
            jax.ShapeDtypeStruct((2, NC, n, WH), jnp.float32),
            jax.ShapeDtypeStruct((2, NS, n), jnp.float32),
        ),
        mesh=mesh,
        compiler_params=pltpu.CompilerParams(
            needs_layout_passes=False, use_tc_tiling_on_sc=False),
        scratch_types=[
            pltpu.VMEM((n,), jnp.float32),
            pltpu.VMEM((n,), jnp.float32),
            pltpu.VMEM((n,), jnp.float32),
            pltpu.VMEM((nch, CHUNK), jnp.int32),
            pltpu.VMEM((nch, CHUNK), jnp.int32),
        ] + [pltpu.VMEM((CHUNK, WH), jnp.float32) for _ in range(4)]
        + [pltpu.VMEM_SHARED((n, WH), jnp.float32)]
        + [pltpu.SemaphoreType.DMA for _ in range(4)],
    )
    def k(table_h, al_h, ar_h, src_h, dst_h, acc_out, den_out,
          al_v, ar_v, den_v, src_v, dst_v, rv0, rv1, rv2, rv3, acc_sh,
          g0, g1, g2, g3):
        rows_bufs = (rv0, rv1, rv2, rv3)
        sems = (g0, g1, g2, g3)
        c = lax.axis_index("c")
        s = lax.axis_index("s")
        zeros16 = jnp.zeros((L,), jnp.float32)

        pltpu.sync_copy(al_h, al_v)
        pltpu.sync_copy(ar_h, ar_v)

        for phase in range(2):
            pltpu.sync_copy(src_h.at[phase, s], src_v)
            pltpu.sync_copy(dst_h.at[phase, s], dst_v)

            @pl.when(c == 1)
            def _():
                def xform(kk, _):
                    for g in range(ng):
                        sl = pl.ds(g * L, L)
                        src_v[kk, sl] = src_v[kk, sl] + 1
                    return 0
                lax.fori_loop(0, nch, xform, 0)

            def zden(i, _):
                den_v[pl.ds(i * L, L)] = zeros16
                return 0
            lax.fori_loop(0, n // L, zden, 0)

            def zrow(i, _):
                for j in range(nj):
                    rv0[i, pl.ds(j * L, L)] = zeros16
                return 0
            lax.fori_loop(0, CHUNK, zrow, 0)

            def zacc(i, _):
                pltpu.sync_copy(
                    rv0,
                    acc_sh.at[pl.ds(s * rows_per_tile + i * CHUNK, CHUNK)])
                return 0
            lax.fori_loop(0, nz, zacc, 0)
            if zrem:
                pltpu.sync_copy(
                    rv0.at[pl.ds(0, zrem)],
                    acc_sh.at[pl.ds(s * rows_per_tile + nz * CHUNK, zrem)])
            plsc.subcore_barrier()

            def visit(kk, b):
                rows_v = rows_bufs[b]
                ps = []
                dvs = []
                for g in range(ng):
                    sv = src_v[kk, pl.ds(g * L, L)]
                    node = lax.shift_right_logical(sv, 1)
                    dv = dst_v[kk, pl.ds(g * L, L)]
                    dvs.append(dv)
                    e = (plsc.load_gather(al_v, [node])
                         + plsc.load_gather(ar_v, [dv]))
                    e = jnp.maximum(e, 0.2 * e)
                    ps.append(jnp.exp(e))

                @pl.when(c == phase)
                def _():
                    for g in range(ng):
                        plsc.addupdate_scatter(den_v, [dvs[g]], ps[g])

                pltpu.make_async_copy(
                    table_h.at[src_v.at[kk]], rows_v, sems[b]).wait()
                for g in range(ng):
                    p = ps[g]
                    for t in range(L):
                        pi = p[t]
                        row = g * L + t
                        for j in range(nj):
                            sl = pl.ds(j * L, L)
                            rows_v[row, sl] = rows_v[row, sl] * pi
                pltpu.async_copy(
                    rows_v, acc_sh.at[dst_v.at[kk]], sems[b], add=True)

            for b in range(4):
                pltpu.async_copy(
                    table_h.at[src_v.at[b]], rows_bufs[b], sems[b])

            def ring_body(i, _):
                for b in range(4):
                    kk = i * 4 + b
                    visit(kk, b)
                    b3 = (b + 3) % 4

                    @pl.when((kk >= 1) & (kk + 3 < nch))
                    def _():
                        pltpu.make_async_copy(
                            rows_bufs[b3], acc_sh.at[dst_v.at[kk]],
                            sems[b3]).wait()
                        pltpu.async_copy(
                            table_h.at[src_v.at[kk + 3]], rows_bufs[b3],
                            sems[b3])
                return 0
            lax.fori_loop(0, (nch - 2) // 4, ring_body, 0)

            visit(nch - 2, 0)
            visit(nch - 1, 1)
            for b in range(4):
                pltpu.make_async_copy(
                    rows_bufs[b], acc_sh.at[dst_v.at[0]], sems[b]).wait()

            plsc.subcore_barrier()
            pltpu.sync_copy(
                acc_sh.at[pl.ds(s * rows_per_tile, rows_per_tile)],
                acc_out.at[phase, c, pl.ds(s * rows_per_tile, rows_per_tile)])

            @pl.when(c == phase)
            def _():
                pltpu.sync_copy(den_v, den_out.at[phase, s])

    return k(table2, al, ar, src4, dst4)


def _tc_prep(x, Ws, Wd, a_s2, a_d2, bn):
    n, d = x.shape
    h = Ws.shape[1]
    grid = (n // bn,)

    def body(x_ref, ws_ref, wd_ref, as_ref, ad_ref, tab_ref, al_ref, ar_ref):
        xb = x_ref[...]
        hs = jnp.dot(xb, ws_ref[...], preferred_element_type=jnp.float32)
        tab_ref[...] = hs
        al_ref[...] = jnp.dot(hs, as_ref[...], preferred_element_type=jnp.float32)
        wdad = jnp.dot(wd_ref[...], ad_ref[...], preferred_element_type=jnp.float32)
        ar_ref[...] = jnp.dot(xb, wdad, preferred_element_type=jnp.float32)

    tab, al, ar = pl.pallas_call(
        body,
        grid=grid,
        in_specs=[
            pl.BlockSpec((bn, d), lambda i: (i, 0)),
            pl.BlockSpec((d, h), lambda i: (0, 0)),
            pl.BlockSpec((d, h), lambda i: (0, 0)),
            pl.BlockSpec((h, 1), lambda i: (0, 0)),
            pl.BlockSpec((h, 1), lambda i: (0, 0)),
        ],
        out_specs=[
            pl.BlockSpec((bn, h), lambda i: (i, 0)),
            pl.BlockSpec((bn, 1), lambda i: (i, 0)),
            pl.BlockSpec((bn, 1), lambda i: (i, 0)),
        ],
        out_shape=[
            jax.ShapeDtypeStruct((n, h), jnp.float32),
            jax.ShapeDtypeStruct((n, 1), jnp.float32),
            jax.ShapeDtypeStruct((n, 1), jnp.float32),
        ],
    )(x, Ws, Wd, a_s2, a_d2)
    return tab.reshape(2 * n, WH), al.reshape(n), ar.reshape(n)


def _combine(acc, den, b2d, eps=1e-16):
    outs = []
    for st in range(2):
        d = jnp.sum(den[st], axis=1)[:, None] + eps
        outs.append(jnp.concatenate(
            [acc[st, hv] / d for hv in range(NC)], axis=1))
    return outs[0] + outs[1] + 2.0 * b2d


def _tc_mid(acc, den, b1_2d, lin1_W, lin1_b2d, W2s, W2d, a2s2, a2d2, n, bn):
    h = 2 * WH
    grid = (n // bn,)

    def body(acc_ref, den_ref, b1_ref, l1w_ref, l1b_ref, w2s_ref, w2d_ref,
             a2s_ref, a2d_ref, tab_ref, al_ref, ar_ref):
        hcomb = _combine(acc_ref[...], den_ref[...], b1_ref[...])
        hh = jnp.dot(hcomb, l1w_ref[...], preferred_element_type=jnp.float32)
        hh = jnp.maximum(hh + l1b_ref[...], 0.0)
        hs = jnp.dot(hh, w2s_ref[...], preferred_element_type=jnp.float32)
        tab_ref[...] = hs
        al_ref[...] = jnp.dot(hs, a2s_ref[...], preferred_element_type=jnp.float32)
        wdad = jnp.dot(w2d_ref[...], a2d_ref[...], preferred_element_type=jnp.float32)
        ar_ref[...] = jnp.dot(hh, wdad, preferred_element_type=jnp.float32)

    tab, al, ar = pl.pallas_call(
        body,
        grid=grid,
        in_specs=[
            pl.BlockSpec((2, NC, bn, WH), lambda i: (0, 0, i, 0)),
            pl.BlockSpec((2, bn, NS), lambda i: (0, i, 0)),
            pl.BlockSpec((1, h), lambda i: (0, 0)),
            pl.BlockSpec((h, h), lambda i: (0, 0)),
            pl.BlockSpec((1, h), lambda i: (0, 0)),
            pl.BlockSpec((h, h), lambda i: (0, 0)),
            pl.BlockSpec((h, h), lambda i: (0, 0)),
            pl.BlockSpec((h, 1), lambda i: (0, 0)),
            pl.BlockSpec((h, 1), lambda i: (0, 0)),
        ],
        out_specs=[
            pl.BlockSpec((bn, h), lambda i: (i, 0)),
            pl.BlockSpec((bn, 1), lambda i: (i, 0)),
            pl.BlockSpec((bn, 1), lambda i: (i, 0)),
        ],
        out_shape=[
            jax.ShapeDtypeStruct((n, h), jnp.float32),
            jax.ShapeDtypeStruct((n, 1), jnp.float32),
            jax.ShapeDtypeStruct((n, 1), jnp.float32),
        ],
    )(acc, den, b1_2d, lin1_W, lin1_b2d, W2s, W2d, a2s2, a2d2)
    return tab.reshape(2 * n, WH), al.reshape(n), ar.reshape(n)


def _tc_final(acc, den, b2_2d, lin2_W, lin2_b2d, n, bn):
    h = 2 * WH
    grid = (n // bn,)

    def body(acc_ref, den_ref, b2_ref, l2w_ref, l2b_ref, out_ref):
        hcomb = _combine(acc_ref[...], den_ref[...], b2_ref[...])
        out_ref[...] = jnp.dot(
            hcomb, l2w_ref[...], preferred_element_type=jnp.float32) + l2b_ref[...]

    return pl.pallas_call(
        body,
        grid=grid,
        in_specs=[
            pl.BlockSpec((2, NC, bn, WH), lambda i: (0, 0, i, 0)),
            pl.BlockSpec((2, bn, NS), lambda i: (0, i, 0)),
            pl.BlockSpec((1, h), lambda i: (0, 0)),
            pl.BlockSpec((h, h), lambda i: (0, 0)),
            pl.BlockSpec((1, h), lambda i: (0, 0)),
        ],
        out_specs=pl.BlockSpec((bn, h), lambda i: (i, 0)),
        out_shape=jax.ShapeDtypeStruct((n, h), jnp.float32),
    )(acc, den, b2_2d, lin2_W, lin2_b2d)


def kernel(x, edge_index, edge_index_2_hop, W1s, W1d, a1s, a1d, b1,
           lin1_W, lin1_b, W2s, W2d, a2s, a2d, b2, lin2_W, lin2_b):
    n, d = x.shape
    e = edge_index.shape[1]
    per_tile = e // NS
    nch = per_tile // CHUNK
    bn = 1000

    src4 = (jnp.stack([edge_index[0], edge_index_2_hop[0]]) * 2).reshape(
        2, NS, nch, CHUNK)
    dst4 = jnp.stack([edge_index[1], edge_index_2_hop[1]]).reshape(
        2, NS, nch, CHUNK)

    tab1, al1, ar1 = _tc_prep(x, W1s, W1d, a1s.reshape(-1, 1),
                              a1d.reshape(-1, 1), bn)
    acc1, den1 = _sc_edge_pass(tab1, al1, ar1, src4, dst4)
    den1 = den1.transpose(0, 2, 1)
    tab2, al2, ar2 = _tc_mid(acc1, den1, b1.reshape(1, -1), lin1_W,
                             lin1_b.reshape(1, -1), W2s, W2d,
                             a2s.reshape(-1, 1), a2d.reshape(-1, 1), n, bn)
    acc2, den2 = _sc_edge_pass(tab2, al2, ar2, src4, dst4)
    den2 = den2.transpose(0, 2, 1)
    return _tc_final(acc2, den2, b2.reshape(1, -1), lin2_W,
                     lin2_b.reshape(1, -1), n, bn)


# speedup vs baseline: 1.0412x; 1.0412x over previous
import functools

import jax
import jax.numpy as jnp
from jax import lax
from jax.experimental import pallas as pl
from jax.experimental.pallas import tpu as pltpu
from jax.experimental.pallas import tpu_sc as plsc

NC = 2
NS = 16
L = 16

CHUNK = 80
WH = 64


def _sc_edge_pass(table2, al, ar, src4, dst4):
    n = table2.shape[0] // 2
    nch = src4.shape[2]
    rows_per_tile = n // NS
    nz = rows_per_tile // CHUNK
    zrem = rows_per_tile % CHUNK
    nj = WH // L
    ng = CHUNK // L

    mesh = plsc.VectorSubcoreMesh(
        core_axis_name="c", subcore_axis_name="s",
        num_cores=NC, num_subcores=NS)

    @functools.partial(
        pl.kernel,
        out_type=(
            jax.ShapeDtypeStruct((2, NC, n, WH), jnp.float32),
            jax.ShapeDtypeStruct((2, NS, n), jnp.float32),
        ),
        mesh=mesh,
        compiler_params=pltpu.CompilerParams(
            needs_layout_passes=False, use_tc_tiling_on_sc=False),
        scratch_types=[
            pltpu.VMEM((n,), jnp.float32),
            pltpu.VMEM((n,), jnp.float32),
            pltpu.VMEM((n,), jnp.float32),
            pltpu.VMEM((nch, CHUNK), jnp.int32),
            pltpu.VMEM((nch, CHUNK), jnp.int32),
        ] + [pltpu.VMEM((CHUNK, WH), jnp.float32) for _ in range(4)]
        + [pltpu.VMEM_SHARED((n, WH), jnp.float32)]
        + [pltpu.SemaphoreType.DMA for _ in range(4)],
    )
    def k(table_h, al_h, ar_h, src_h, dst_h, acc_out, den_out,
          al_v, ar_v, den_v, src_v, dst_v, rv0, rv1, rv2, rv3, acc_sh,
          g0, g1, g2, g3):
        rows_bufs = (rv0, rv1, rv2, rv3)
        sems = (g0, g1, g2, g3)
        c = lax.axis_index("c")
        s = lax.axis_index("s")
        zeros16 = jnp.zeros((L,), jnp.float32)

        pltpu.sync_copy(al_h, al_v)
        pltpu.sync_copy(ar_h, ar_v)

        for phase in range(2):
            pltpu.sync_copy(src_h.at[phase, s], src_v)
            pltpu.sync_copy(dst_h.at[phase, s], dst_v)

            def xform(kk, _):
                for g in range(ng):
                    sl = pl.ds(g * L, L)
                    src_v[kk, sl] = src_v[kk, sl] * 2 + c
                return 0
            lax.fori_loop(0, nch, xform, 0)

            def zden(i, _):
                den_v[pl.ds(i * L, L)] = zeros16
                return 0
            lax.fori_loop(0, n // L, zden, 0)

            def zrow(i, _):
                for j in range(nj):
                    rv0[i, pl.ds(j * L, L)] = zeros16
                return 0
            lax.fori_loop(0, CHUNK, zrow, 0)

            def zacc(i, _):
                pltpu.sync_copy(
                    rv0,
                    acc_sh.at[pl.ds(s * rows_per_tile + i * CHUNK, CHUNK)])
                return 0
            lax.fori_loop(0, nz, zacc, 0)
            if zrem:
                pltpu.sync_copy(
                    rv0.at[pl.ds(0, zrem)],
                    acc_sh.at[pl.ds(s * rows_per_tile + nz * CHUNK, zrem)])
            plsc.subcore_barrier()

            def visit(kk, b):
                rows_v = rows_bufs[b]
                ps = []
                for g in range(ng):
                    sv = src_v[kk, pl.ds(g * L, L)]
                    node = lax.shift_right_logical(sv, 1)
                    dv = dst_v[kk, pl.ds(g * L, L)]
                    e = (plsc.load_gather(al_v, [node])
                         + plsc.load_gather(ar_v, [dv]))
                    e = jnp.maximum(e, 0.2 * e)
                    p = jnp.exp(e)
                    ps.append(p)

                    @pl.when(c == 0)
                    def _():
                        plsc.addupdate_scatter(den_v, [dv], p)

                pltpu.make_async_copy(
                    table_h.at[src_v.at[kk]], rows_v, sems[b]).wait()
                for g in range(ng):
                    p = ps[g]
                    for t in range(L):
                        pi = p[t]
                        row = g * L + t
                        for j in range(nj):
                            sl = pl.ds(j * L, L)
                            rows_v[row, sl] = rows_v[row, sl] * pi
                pltpu.async_copy(
                    rows_v, acc_sh.at[dst_v.at[kk]], sems[b], add=True)

            for b in range(4):
                pltpu.async_copy(
                    table_h.at[src_v.at[b]], rows_bufs[b], sems[b])

            def ring_body(i, _):
                for b in range(4):
                    kk = i * 4 + b
                    visit(kk, b)
                    b3 = (b + 3) % 4

                    @pl.when((kk >= 1) & (kk + 3 < nch))
                    def _():
                        pltpu.make_async_copy(
                            rows_bufs[b3], acc_sh.at[dst_v.at[kk]],
                            sems[b3]).wait()
                        pltpu.async_copy(
                            table_h.at[src_v.at[kk + 3]], rows_bufs[b3],
                            sems[b3])
                return 0
            lax.fori_loop(0, (nch - 2) // 4, ring_body, 0)

            visit(nch - 2, 0)
            visit(nch - 1, 1)
            for b in range(4):
                pltpu.make_async_copy(
                    rows_bufs[b], acc_sh.at[dst_v.at[0]], sems[b]).wait()

            plsc.subcore_barrier()
            pltpu.sync_copy(
                acc_sh.at[pl.ds(s * rows_per_tile, rows_per_tile)],
                acc_out.at[phase, c, pl.ds(s * rows_per_tile, rows_per_tile)])

            @pl.when(c == 0)
            def _():
                pltpu.sync_copy(den_v, den_out.at[phase, s])
            plsc.subcore_barrier()

    return k(table2, al, ar, src4, dst4)


def _tc_prep(x, Ws, Wd, a_s2, a_d2, bn):
    n, d = x.shape
    h = Ws.shape[1]
    grid = (n // bn,)

    def body(x_ref, ws_ref, wd_ref, as_ref, ad_ref, tab_ref, al_ref, ar_ref):
        xb = x_ref[...]
        hs = jnp.dot(xb, ws_ref[...], preferred_element_type=jnp.float32)
        tab_ref[...] = hs
        al_ref[...] = jnp.dot(hs, as_ref[...], preferred_element_type=jnp.float32)
        wdad = jnp.dot(wd_ref[...], ad_ref[...], preferred_element_type=jnp.float32)
        ar_ref[...] = jnp.dot(xb, wdad, preferred_element_type=jnp.float32)

    tab, al, ar = pl.pallas_call(
        body,
        grid=grid,
        in_specs=[
            pl.BlockSpec((bn, d), lambda i: (i, 0)),
            pl.BlockSpec((d, h), lambda i: (0, 0)),
            pl.BlockSpec((d, h), lambda i: (0, 0)),
            pl.BlockSpec((h, 1), lambda i: (0, 0)),
            pl.BlockSpec((h, 1), lambda i: (0, 0)),
        ],
        out_specs=[
            pl.BlockSpec((bn, h), lambda i: (i, 0)),
            pl.BlockSpec((bn, 1), lambda i: (i, 0)),
            pl.BlockSpec((bn, 1), lambda i: (i, 0)),
        ],
        out_shape=[
            jax.ShapeDtypeStruct((n, h), jnp.float32),
            jax.ShapeDtypeStruct((n, 1), jnp.float32),
            jax.ShapeDtypeStruct((n, 1), jnp.float32),
        ],
    )(x, Ws, Wd, a_s2, a_d2)
    return tab.reshape(2 * n, WH), al.reshape(n), ar.reshape(n)


def _combine(acc, den, b2d, eps=1e-16):
    outs = []
    for st in range(2):
        d = jnp.sum(den[st], axis=1)[:, None] + eps
        outs.append(jnp.concatenate(
            [acc[st, hv] / d for hv in range(NC)], axis=1))
    return outs[0] + outs[1] + 2.0 * b2d


def _tc_mid(acc, den, b1_2d, lin1_W, lin1_b2d, W2s, W2d, a2s2, a2d2, n, bn):
    h = 2 * WH
    grid = (n // bn,)

    def body(acc_ref, den_ref, b1_ref, l1w_ref, l1b_ref, w2s_ref, w2d_ref,
             a2s_ref, a2d_ref, tab_ref, al_ref, ar_ref):
        hcomb = _combine(acc_ref[...], den_ref[...], b1_ref[...])
        hh = jnp.dot(hcomb, l1w_ref[...], preferred_element_type=jnp.float32)
        hh = jnp.maximum(hh + l1b_ref[...], 0.0)
        hs = jnp.dot(hh, w2s_ref[...], preferred_element_type=jnp.float32)
        tab_ref[...] = hs
        al_ref[...] = jnp.dot(hs, a2s_ref[...], preferred_element_type=jnp.float32)
        wdad = jnp.dot(w2d_ref[...], a2d_ref[...], preferred_element_type=jnp.float32)
        ar_ref[...] = jnp.dot(hh, wdad, preferred_element_type=jnp.float32)

    tab, al, ar = pl.pallas_call(
        body,
        grid=grid,
        in_specs=[
            pl.BlockSpec((2, NC, bn, WH), lambda i: (0, 0, i, 0)),
            pl.BlockSpec((2, bn, NS), lambda i: (0, i, 0)),
            pl.BlockSpec((1, h), lambda i: (0, 0)),
            pl.BlockSpec((h, h), lambda i: (0, 0)),
            pl.BlockSpec((1, h), lambda i: (0, 0)),
            pl.BlockSpec((h, h), lambda i: (0, 0)),
            pl.BlockSpec((h, h), lambda i: (0, 0)),
            pl.BlockSpec((h, 1), lambda i: (0, 0)),
            pl.BlockSpec((h, 1), lambda i: (0, 0)),
        ],
        out_specs=[
            pl.BlockSpec((bn, h), lambda i: (i, 0)),
            pl.BlockSpec((bn, 1), lambda i: (i, 0)),
            pl.BlockSpec((bn, 1), lambda i: (i, 0)),
        ],
        out_shape=[
            jax.ShapeDtypeStruct((n, h), jnp.float32),
            jax.ShapeDtypeStruct((n, 1), jnp.float32),
            jax.ShapeDtypeStruct((n, 1), jnp.float32),
        ],
    )(acc, den, b1_2d, lin1_W, lin1_b2d, W2s, W2d, a2s2, a2d2)
    return tab.reshape(2 * n, WH), al.reshape(n), ar.reshape(n)


def _tc_final(acc, den, b2_2d, lin2_W, lin2_b2d, n, bn):
    h = 2 * WH
    grid = (n // bn,)

    def body(acc_ref, den_ref, b2_ref, l2w_ref, l2b_ref, out_ref):
        hcomb = _combine(acc_ref[...], den_ref[...], b2_ref[...])
        out_ref[...] = jnp.dot(
            hcomb, l2w_ref[...], preferred_element_type=jnp.float32) + l2b_ref[...]

    return pl.pallas_call(
        body,
        grid=grid,
        in_specs=[
            pl.BlockSpec((2, NC, bn, WH), lambda i: (0, 0, i, 0)),
            pl.BlockSpec((2, bn, NS), lambda i: (0, i, 0)),
            pl.BlockSpec((1, h), lambda i: (0, 0)),
            pl.BlockSpec((h, h), lambda i: (0, 0)),
            pl.BlockSpec((1, h), lambda i: (0, 0)),
        ],
        out_specs=pl.BlockSpec((bn, h), lambda i: (i, 0)),
        out_shape=jax.ShapeDtypeStruct((n, h), jnp.float32),
    )(acc, den, b2_2d, lin2_W, lin2_b2d)


def kernel(x, edge_index, edge_index_2_hop, W1s, W1d, a1s, a1d, b1,
           lin1_W, lin1_b, W2s, W2d, a2s, a2d, b2, lin2_W, lin2_b):
    n, d = x.shape
    e = edge_index.shape[1]
    per_tile = e // NS
    nch = per_tile // CHUNK
    bn = 1000

    src4 = jnp.stack([edge_index[0], edge_index_2_hop[0]]).reshape(
        2, NS, nch, CHUNK)
    dst4 = jnp.stack([edge_index[1], edge_index_2_hop[1]]).reshape(
        2, NS, nch, CHUNK)

    tab1, al1, ar1 = _tc_prep(x, W1s, W1d, a1s.reshape(-1, 1),
                              a1d.reshape(-1, 1), bn)
    acc1, den1 = _sc_edge_pass(tab1, al1, ar1, src4, dst4)
    den1 = den1.transpose(0, 2, 1)
    tab2, al2, ar2 = _tc_mid(acc1, den1, b1.reshape(1, -1), lin1_W,
                             lin1_b.reshape(1, -1), W2s, W2d,
                             a2s.reshape(-1, 1), a2d.reshape(-1, 1), n, bn)
    acc2, den2 = _sc_edge_pass(tab2, al2, ar2, src4, dst4)
    den2 = den2.transpose(0, 2, 1)
    return _tc_final(acc2, den2, b2.reshape(1, -1), lin2_W,
                     lin2_b.reshape(1, -1), n, bn)


# speedup vs baseline: 1.0456x; 1.0042x over previous
import functools

import jax
import jax.numpy as jnp
from jax import lax
from jax.experimental import pallas as pl
from jax.experimental.pallas import tpu as pltpu
from jax.experimental.pallas import tpu_sc as plsc

NC = 2
NS = 16
L = 16

CHUNK = 80
WH = 64


def _sc_edge_pass(table2, al, ar, src4, dst4):
    n = table2.shape[0] // 2
    nch = src4.shape[2]
    rows_per_tile = n // NS
    nz = rows_per_tile // CHUNK
    zrem = rows_per_tile % CHUNK
    nj = WH // L
    ng = CHUNK // L

    mesh = plsc.VectorSubcoreMesh(
        core_axis_name="c", subcore_axis_name="s",
        num_cores=NC, num_subcores=NS)

    @functools.partial(
        pl.kernel,
        out_type=(
            jax.ShapeDtypeStruct((2, NC, n, WH), jnp.float32),
            jax.ShapeDtypeStruct((2, NS, n), jnp.float32),
        ),
        mesh=mesh,
        compiler_params=pltpu.CompilerParams(
            needs_layout_passes=False, use_tc_tiling_on_sc=False),
        scratch_types=[
            pltpu.VMEM((n,), jnp.float32),
            pltpu.VMEM((n,), jnp.float32),
            pltpu.VMEM((n,), jnp.float32),
            pltpu.VMEM((nch, CHUNK), jnp.int32),
            pltpu.VMEM((nch, CHUNK), jnp.int32),
        ] + [pltpu.VMEM((CHUNK, WH), jnp.float32) for _ in range(4)]
        + [pltpu.VMEM_SHARED((n, WH), jnp.float32)]
        + [pltpu.SemaphoreType.DMA for _ in range(4)],
    )
    def k(table_h, al_h, ar_h, src_h, dst_h, acc_out, den_out,
          al_v, ar_v, den_v, src_v, dst_v, rv0, rv1, rv2, rv3, acc_sh,
          g0, g1, g2, g3):
        rows_bufs = (rv0, rv1, rv2, rv3)
        sems = (g0, g1, g2, g3)
        c = lax.axis_index("c")
        s = lax.axis_index("s")
        zeros16 = jnp.zeros((L,), jnp.float32)

        pltpu.sync_copy(al_h, al_v)
        pltpu.sync_copy(ar_h, ar_v)

        for phase in range(2):
            cp_src = pltpu.async_copy(src_h.at[phase, s], src_v, g0)
            cp_dst = pltpu.async_copy(dst_h.at[phase, s], dst_v, g1)
            cp_src.wait()
            cp_dst.wait()

            def xform(kk, _):
                for g in range(ng):
                    sl = pl.ds(g * L, L)
                    src_v[kk, sl] = src_v[kk, sl] * 2 + c
                return 0
            lax.fori_loop(0, nch, xform, 0)

            def zden(i, _):
                den_v[pl.ds(i * L, L)] = zeros16
                return 0
            lax.fori_loop(0, n // L, zden, 0)

            def zrow(i, _):
                for j in range(nj):
                    rv0[i, pl.ds(j * L, L)] = zeros16
                return 0
            lax.fori_loop(0, CHUNK, zrow, 0)

            zcps = []
            for i in range(nz):
                zcps.append(pltpu.async_copy(
                    rv0,
                    acc_sh.at[pl.ds(s * rows_per_tile + i * CHUNK, CHUNK)],
                    sems[i % 4]))
            if zrem:
                zcps.append(pltpu.async_copy(
                    rv0.at[pl.ds(0, zrem)],
                    acc_sh.at[pl.ds(s * rows_per_tile + nz * CHUNK, zrem)],
                    sems[nz % 4]))
            for cp in zcps:
                cp.wait()
            plsc.subcore_barrier()

            def visit(kk, b):
                rows_v = rows_bufs[b]
                ps = []
                for g in range(ng):
                    sv = src_v[kk, pl.ds(g * L, L)]
                    node = lax.shift_right_logical(sv, 1)
                    dv = dst_v[kk, pl.ds(g * L, L)]
                    e = (plsc.load_gather(al_v, [node])
                         + plsc.load_gather(ar_v, [dv]))
                    e = jnp.maximum(e, 0.2 * e)
                    p = jnp.exp(e)
                    ps.append(p)

                    @pl.when(c == 0)
                    def _():
                        plsc.addupdate_scatter(den_v, [dv], p)

                pltpu.make_async_copy(
                    table_h.at[src_v.at[kk]], rows_v, sems[b]).wait()
                for g in range(ng):
                    p = ps[g]
                    for t in range(L):
                        pi = p[t]
                        row = g * L + t
                        for j in range(nj):
                            sl = pl.ds(j * L, L)
                            rows_v[row, sl] = rows_v[row, sl] * pi
                pltpu.async_copy(
                    rows_v, acc_sh.at[dst_v.at[kk]], sems[b], add=True)

            for b in range(4):
                pltpu.async_copy(
                    table_h.at[src_v.at[b]], rows_bufs[b], sems[b])

            def ring_body(i, _):
                for b in range(4):
                    kk = i * 4 + b
                    visit(kk, b)
                    b3 = (b + 3) % 4

                    @pl.when((kk >= 1) & (kk + 3 < nch))
                    def _():
                        pltpu.make_async_copy(
                            rows_bufs[b3], acc_sh.at[dst_v.at[kk]],
                            sems[b3]).wait()
                        pltpu.async_copy(
                            table_h.at[src_v.at[kk + 3]], rows_bufs[b3],
                            sems[b3])
                return 0
            lax.fori_loop(0, (nch - 2) // 4, ring_body, 0)

            visit(nch - 2, 0)
            visit(nch - 1, 1)
            for b in range(4):
                pltpu.make_async_copy(
                    rows_bufs[b], acc_sh.at[dst_v.at[0]], sems[b]).wait()

            plsc.subcore_barrier()
            pltpu.sync_copy(
                acc_sh.at[pl.ds(s * rows_per_tile, rows_per_tile)],
                acc_out.at[phase, c, pl.ds(s * rows_per_tile, rows_per_tile)])

            @pl.when(c == 0)
            def _():
                pltpu.sync_copy(den_v, den_out.at[phase, s])
            plsc.subcore_barrier()

    return k(table2, al, ar, src4, dst4)


def _tc_prep(x, Ws, Wd, a_s2, a_d2, bn):
    n, d = x.shape
    h = Ws.shape[1]
    grid = (n // bn,)

    def body(x_ref, ws_ref, wd_ref, as_ref, ad_ref, tab_ref, al_ref, ar_ref):
        xb = x_ref[...]
        hs = jnp.dot(xb, ws_ref[...], preferred_element_type=jnp.float32)
        tab_ref[...] = hs
        al_ref[...] = jnp.dot(hs, as_ref[...], preferred_element_type=jnp.float32)
        wdad = jnp.dot(wd_ref[...], ad_ref[...], preferred_element_type=jnp.float32)
        ar_ref[...] = jnp.dot(xb, wdad, preferred_element_type=jnp.float32)

    tab, al, ar = pl.pallas_call(
        body,
        grid=grid,
        in_specs=[
            pl.BlockSpec((bn, d), lambda i: (i, 0)),
            pl.BlockSpec((d, h), lambda i: (0, 0)),
            pl.BlockSpec((d, h), lambda i: (0, 0)),
            pl.BlockSpec((h, 1), lambda i: (0, 0)),
            pl.BlockSpec((h, 1), lambda i: (0, 0)),
        ],
        out_specs=[
            pl.BlockSpec((bn, h), lambda i: (i, 0)),
            pl.BlockSpec((bn, 1), lambda i: (i, 0)),
            pl.BlockSpec((bn, 1), lambda i: (i, 0)),
        ],
        out_shape=[
            jax.ShapeDtypeStruct((n, h), jnp.float32),
            jax.ShapeDtypeStruct((n, 1), jnp.float32),
            jax.ShapeDtypeStruct((n, 1), jnp.float32),
        ],
    )(x, Ws, Wd, a_s2, a_d2)
    return tab.reshape(2 * n, WH), al.reshape(n), ar.reshape(n)


def _combine(acc, den, b2d, eps=1e-16):
    outs = []
    for st in range(2):
        d = jnp.sum(den[st], axis=1)[:, None] + eps
        outs.append(jnp.concatenate(
            [acc[st, hv] / d for hv in range(NC)], axis=1))
    return outs[0] + outs[1] + 2.0 * b2d


def _tc_mid(acc, den, b1_2d, lin1_W, lin1_b2d, W2s, W2d, a2s2, a2d2, n, bn):
    h = 2 * WH
    grid = (n // bn,)

    def body(acc_ref, den_ref, b1_ref, l1w_ref, l1b_ref, w2s_ref, w2d_ref,
             a2s_ref, a2d_ref, tab_ref, al_ref, ar_ref):
        hcomb = _combine(acc_ref[...], den_ref[...], b1_ref[...])
        hh = jnp.dot(hcomb, l1w_ref[...], preferred_element_type=jnp.float32)
        hh = jnp.maximum(hh + l1b_ref[...], 0.0)
        hs = jnp.dot(hh, w2s_ref[...], preferred_element_type=jnp.float32)
        tab_ref[...] = hs
        al_ref[...] = jnp.dot(hs, a2s_ref[...], preferred_element_type=jnp.float32)
        wdad = jnp.dot(w2d_ref[...], a2d_ref[...], preferred_element_type=jnp.float32)
        ar_ref[...] = jnp.dot(hh, wdad, preferred_element_type=jnp.float32)

    tab, al, ar = pl.pallas_call(
        body,
        grid=grid,
        in_specs=[
            pl.BlockSpec((2, NC, bn, WH), lambda i: (0, 0, i, 0)),
            pl.BlockSpec((2, bn, NS), lambda i: (0, i, 0)),
            pl.BlockSpec((1, h), lambda i: (0, 0)),
            pl.BlockSpec((h, h), lambda i: (0, 0)),
            pl.BlockSpec((1, h), lambda i: (0, 0)),
            pl.BlockSpec((h, h), lambda i: (0, 0)),
            pl.BlockSpec((h, h), lambda i: (0, 0)),
            pl.BlockSpec((h, 1), lambda i: (0, 0)),
            pl.BlockSpec((h, 1), lambda i: (0, 0)),
        ],
        out_specs=[
            pl.BlockSpec((bn, h), lambda i: (i, 0)),
            pl.BlockSpec((bn, 1), lambda i: (i, 0)),
            pl.BlockSpec((bn, 1), lambda i: (i, 0)),
        ],
        out_shape=[
            jax.ShapeDtypeStruct((n, h), jnp.float32),
            jax.ShapeDtypeStruct((n, 1), jnp.float32),
            jax.ShapeDtypeStruct((n, 1), jnp.float32),
        ],
    )(acc, den, b1_2d, lin1_W, lin1_b2d, W2s, W2d, a2s2, a2d2)
    return tab.reshape(2 * n, WH), al.reshape(n), ar.reshape(n)


def _tc_final(acc, den, b2_2d, lin2_W, lin2_b2d, n, bn):
    h = 2 * WH
    grid = (n // bn,)

    def body(acc_ref, den_ref, b2_ref, l2w_ref, l2b_ref, out_ref):
        hcomb = _combine(acc_ref[...], den_ref[...], b2_ref[...])
        out_ref[...] = jnp.dot(
            hcomb, l2w_ref[...], preferred_element_type=jnp.float32) + l2b_ref[...]

    return pl.pallas_call(
        body,
        grid=grid,
        in_specs=[
            pl.BlockSpec((2, NC, bn, WH), lambda i: (0, 0, i, 0)),
            pl.BlockSpec((2, bn, NS), lambda i: (0, i, 0)),
            pl.BlockSpec((1, h), lambda i: (0, 0)),
            pl.BlockSpec((h, h), lambda i: (0, 0)),
            pl.BlockSpec((1, h), lambda i: (0, 0)),
        ],
        out_specs=pl.BlockSpec((bn, h), lambda i: (i, 0)),
        out_shape=jax.ShapeDtypeStruct((n, h), jnp.float32),
    )(acc, den, b2_2d, lin2_W, lin2_b2d)


def kernel(x, edge_index, edge_index_2_hop, W1s, W1d, a1s, a1d, b1,
           lin1_W, lin1_b, W2s, W2d, a2s, a2d, b2, lin2_W, lin2_b):
    n, d = x.shape
    e = edge_index.shape[1]
    per_tile = e // NS
    nch = per_tile // CHUNK
    bn = 1000

    src4 = jnp.stack([edge_index[0], edge_index_2_hop[0]]).reshape(
        2, NS, nch, CHUNK)
    dst4 = jnp.stack([edge_index[1], edge_index_2_hop[1]]).reshape(
        2, NS, nch, CHUNK)

    tab1, al1, ar1 = _tc_prep(x, W1s, W1d, a1s.reshape(-1, 1),
                              a1d.reshape(-1, 1), bn)
    acc1, den1 = _sc_edge_pass(tab1, al1, ar1, src4, dst4)
    den1 = den1.transpose(0, 2, 1)
    tab2, al2, ar2 = _tc_mid(acc1, den1, b1.reshape(1, -1), lin1_W,
                             lin1_b.reshape(1, -1), W2s, W2d,
                             a2s.reshape(-1, 1), a2d.reshape(-1, 1), n, bn)
    acc2, den2 = _sc_edge_pass(tab2, al2, ar2, src4, dst4)
    den2 = den2.transpose(0, 2, 1)
    return _tc_final(acc2, den2, b2.reshape(1, -1), lin2_W,
                     lin2_b.reshape(1, -1), n, bn)
